# Initial kernel scaffold; baseline (speedup 1.0000x reference)
import jax
import jax.numpy as jnp
from jax.experimental import pallas as pl


def kernel(target_emb, candidate_emb, in_proj_weight, in_proj_bias, out_proj_weight, out_proj_bias):
    raise NotImplementedError("write your pallas kernel here")


# speedup vs baseline: 5.1413x; 5.1413x over previous
import numpy as np
import jax
import jax.numpy as jnp
from jax.experimental import pallas as pl
from jax.experimental.pallas import tpu as pltpu

TAU_ = 1.0
THRESHOLD_ = 0.2


def _make_body(num_chunks, chunk, N, S, E):
    scale = 1.0 / np.sqrt(E)

    def body(cand_ref, tgt_ref, win_ref, bin_ref, wout_ref, bout_ref, u_ref,
             out_ref, mask_ref, qs_ref, lb_ref, acc_ref):
        i = pl.program_id(0)

        @pl.when(i == 0)
        def _prologue():
            t = tgt_ref[...]
            wq = win_ref[0:E, :]
            wk = win_ref[E:2 * E, :]
            q = jnp.dot(t, wq.T, preferred_element_type=jnp.float32) + bin_ref[0:1, :]
            qs_ref[...] = jnp.dot(q * scale, wk, preferred_element_type=jnp.float32)
            acc_ref[...] = jnp.zeros_like(acc_ref)

        x3 = cand_ref[...].reshape(chunk, N, E)
        qs = qs_ref[...]
        lg = jnp.sum(x3 * qs[None, :, :], axis=2)
        lb_ref[:, pl.ds(i * chunk, chunk)] = lg.T
        e = jnp.exp(lg)
        acc_ref[...] += jnp.sum(x3 * e[:, :, None], axis=0)

        @pl.when(i == num_chunks - 1)
        def _epilogue():
            lb = lb_ref[...]
            mx = jnp.max(lb, axis=1, keepdims=True)
            a = jnp.exp(lb - mx)
            ssum = jnp.sum(a, axis=1, keepdims=True)
            a = a / ssum
            g = -jnp.log(-jnp.log(u_ref[...]))
            z = (a + g) / TAU_
            zm = jnp.max(z, axis=1, keepdims=True)
            e2 = jnp.exp(z - zm)
            y = e2 / jnp.sum(e2, axis=1, keepdims=True)
            mask_ref[...] = (y > THRESHOLD_).astype(jnp.int8)
            wctx = acc_ref[...] * (jnp.exp(-mx) / ssum)
            wv = win_ref[2 * E:3 * E, :]
            ctx = jnp.dot(wctx, wv.T, preferred_element_type=jnp.float32) + bin_ref[2:3, :]
            out_ref[...] = (jnp.dot(ctx, wout_ref[...].T, preferred_element_type=jnp.float32)
                            + bout_ref[...])

    return body


def kernel(target_emb, candidate_emb, in_proj_weight, in_proj_bias,
           out_proj_weight, out_proj_bias, interpret=False):
    S, N, E = candidate_emb.shape
    chunk = 256
    num_chunks = S // chunk

    cand2d = candidate_emb.reshape(S * N, E)
    tgt = target_emb.reshape(N, E)
    bin3 = in_proj_bias.reshape(3, E)
    bout2 = out_proj_bias.reshape(1, E)
    u = jax.random.uniform(jax.random.key(42), (N, 1, S),
                           minval=1e-10, maxval=1.0).reshape(N, S)

    out, mask8 = pl.pallas_call(
        _make_body(num_chunks, chunk, N, S, E),
        grid=(num_chunks,),
        in_specs=[
            pl.BlockSpec((chunk * N, E), lambda i: (i, 0)),
            pl.BlockSpec((N, E), lambda i: (0, 0)),
            pl.BlockSpec((3 * E, E), lambda i: (0, 0)),
            pl.BlockSpec((3, E), lambda i: (0, 0)),
            pl.BlockSpec((E, E), lambda i: (0, 0)),
            pl.BlockSpec((1, E), lambda i: (0, 0)),
            pl.BlockSpec((N, S), lambda i: (0, 0)),
        ],
        out_specs=[
            pl.BlockSpec((N, E), lambda i: (0, 0)),
            pl.BlockSpec((N, S), lambda i: (0, 0)),
        ],
        out_shape=[
            jax.ShapeDtypeStruct((N, E), jnp.float32),
            jax.ShapeDtypeStruct((N, S), jnp.int8),
        ],
        scratch_shapes=[
            pltpu.VMEM((N, E), jnp.float32),
            pltpu.VMEM((N, S), jnp.float32),
            pltpu.VMEM((N, E), jnp.float32),
        ],
        compiler_params=pltpu.CompilerParams(
            dimension_semantics=("arbitrary",),
        ),
        interpret=interpret,
    )(cand2d, tgt, in_proj_weight, bin3, out_proj_weight, bout2, u)

    return out, mask8.astype(jnp.bool_).reshape(N, 1, S)


# speedup vs baseline: 5.5538x; 1.0802x over previous
import numpy as np
import jax
import jax.numpy as jnp
from jax.experimental import pallas as pl
from jax.experimental.pallas import tpu as pltpu

TAU_ = 1.0
THRESHOLD_ = 0.2


def _make_body(num_chunks, chunk, N, S, E):
    scale = 1.0 / np.sqrt(E)

    def body(cand_ref, tgt_ref, win_ref, bin_ref, wout_ref, bout_ref, u_ref,
             out_ref, mask_ref, qs_ref, lb_ref, acc_ref):
        i = pl.program_id(0)

        @pl.when(i == 0)
        def _prologue():
            t = tgt_ref[...]
            wq = win_ref[0:E, :]
            wk = win_ref[E:2 * E, :]
            q = jnp.dot(t, wq.T, preferred_element_type=jnp.float32) + bin_ref[0:1, :]
            qs_ref[...] = jnp.dot(q * scale, wk, preferred_element_type=jnp.float32)
            acc_ref[...] = jnp.zeros_like(acc_ref)

        x3 = cand_ref[...].reshape(chunk, N, E)
        qs = qs_ref[...]
        lg = jnp.sum(x3 * qs[None, :, :], axis=2)
        lb_ref[:, pl.ds(i * chunk, chunk)] = lg.T
        e = jnp.exp(lg)
        acc_ref[...] += jnp.sum(x3 * e[:, :, None], axis=0)

        @pl.when(i == num_chunks - 1)
        def _epilogue():
            lb = lb_ref[...]
            mx = jnp.max(lb, axis=1, keepdims=True)
            a = jnp.exp(lb - mx)
            ssum = jnp.sum(a, axis=1, keepdims=True)
            a = a / ssum
            g = -jnp.log(-jnp.log(u_ref[...]))
            z = (a + g) / TAU_
            zm = jnp.max(z, axis=1, keepdims=True)
            e2 = jnp.exp(z - zm)
            y = e2 / jnp.sum(e2, axis=1, keepdims=True)
            mask_ref[...] = (y > THRESHOLD_).astype(jnp.int8)
            wctx = acc_ref[...] * (jnp.exp(-mx) / ssum)
            wv = win_ref[2 * E:3 * E, :]
            ctx = jnp.dot(wctx, wv.T, preferred_element_type=jnp.float32) + bin_ref[2:3, :]
            out_ref[...] = (jnp.dot(ctx, wout_ref[...].T, preferred_element_type=jnp.float32)
                            + bout_ref[...])

    return body


def kernel(target_emb, candidate_emb, in_proj_weight, in_proj_bias,
           out_proj_weight, out_proj_bias, interpret=False):
    S, N, E = candidate_emb.shape
    chunk = 512
    num_chunks = S // chunk

    cand2d = candidate_emb.reshape(S * N, E)
    tgt = target_emb.reshape(N, E)
    bin3 = in_proj_bias.reshape(3, E)
    bout2 = out_proj_bias.reshape(1, E)
    u = jax.random.uniform(jax.random.key(42), (N, 1, S),
                           minval=1e-10, maxval=1.0).reshape(N, S)

    out, mask8 = pl.pallas_call(
        _make_body(num_chunks, chunk, N, S, E),
        grid=(num_chunks,),
        in_specs=[
            pl.BlockSpec((chunk * N, E), lambda i: (i, 0)),
            pl.BlockSpec((N, E), lambda i: (0, 0)),
            pl.BlockSpec((3 * E, E), lambda i: (0, 0)),
            pl.BlockSpec((3, E), lambda i: (0, 0)),
            pl.BlockSpec((E, E), lambda i: (0, 0)),
            pl.BlockSpec((1, E), lambda i: (0, 0)),
            pl.BlockSpec((N, S), lambda i: (0, 0)),
        ],
        out_specs=[
            pl.BlockSpec((N, E), lambda i: (0, 0)),
            pl.BlockSpec((N, S), lambda i: (0, 0)),
        ],
        out_shape=[
            jax.ShapeDtypeStruct((N, E), jnp.float32),
            jax.ShapeDtypeStruct((N, S), jnp.int8),
        ],
        scratch_shapes=[
            pltpu.VMEM((N, E), jnp.float32),
            pltpu.VMEM((N, S), jnp.float32),
            pltpu.VMEM((N, E), jnp.float32),
        ],
        compiler_params=pltpu.CompilerParams(
            dimension_semantics=("arbitrary",),
        ),
        interpret=interpret,
    )(cand2d, tgt, in_proj_weight, bin3, out_proj_weight, bout2, u)

    return out, mask8.astype(jnp.bool_).reshape(N, 1, S)
